# baseline (speedup 1.0000x reference)
import jax
import jax.numpy as jnp
from jax.experimental import pallas as pl
from jax.experimental.pallas import tpu as pltpu

_BN = 4096


def _loss_body(x_ref, r_ref, mu_ref, o_ref):
    x = x_ref[...]
    mu = mu_ref[...]
    x2 = jnp.sum(x * x, axis=1, keepdims=True)
    mu2 = jnp.sum(mu * mu, axis=1, keepdims=True).T
    cross = jax.lax.dot_general(
        x, mu,
        dimension_numbers=(((1,), (1,)), ((), ())),
        preferred_element_type=jnp.float32,
    )
    dist2 = jnp.maximum(x2 + mu2 - 2.0 * cross, 0.0)
    o_ref[0, 0, :] = jnp.sum(r_ref[...] * dist2, axis=0)


def kernel(X, r, mus):
    n, d = X.shape
    m = mus.shape[0]
    g = n // _BN
    g2 = g // 2
    partials = pl.pallas_call(
        _loss_body,
        grid=(2, g2),
        in_specs=[
            pl.BlockSpec((_BN, d), lambda i, j: (i * g2 + j, 0)),
            pl.BlockSpec((_BN, m), lambda i, j: (i * g2 + j, 0)),
            pl.BlockSpec((m, d), lambda i, j: (0, 0)),
        ],
        out_specs=pl.BlockSpec((1, 1, m), lambda i, j: (i * g2 + j, 0, 0)),
        out_shape=jax.ShapeDtypeStruct((g, 1, m), jnp.float32),
        compiler_params=pltpu.CompilerParams(
            dimension_semantics=("parallel", "arbitrary"),
        ),
    )(X, r, mus)
    return jnp.sum(partials) / n


# speedup vs baseline: 1.1929x; 1.1929x over previous
import jax
import jax.numpy as jnp
from jax.experimental import pallas as pl
from jax.experimental.pallas import tpu as pltpu

_BH = 4096


def _half_loss(x, r, mu, mu2):
    x2 = jnp.sum(x * x, axis=1, keepdims=True)
    cross = jax.lax.dot_general(
        x, mu,
        dimension_numbers=(((1,), (1,)), ((), ())),
        preferred_element_type=jnp.float32,
    )
    dist2 = jnp.maximum(x2 + mu2 - 2.0 * cross, 0.0)
    return jnp.sum(r * dist2, axis=0)


def _loss_body(x0_ref, x1_ref, r0_ref, r1_ref, mu_ref, o_ref):
    mu = mu_ref[...]
    mu2 = jnp.sum(mu * mu, axis=1, keepdims=True).T
    s0 = _half_loss(x0_ref[...], r0_ref[...], mu, mu2)
    s1 = _half_loss(x1_ref[...], r1_ref[...], mu, mu2)
    o_ref[0, 0, :] = s0 + s1


def kernel(X, r, mus):
    n, d = X.shape
    m = mus.shape[0]
    g = n // (2 * _BH)
    g2 = g // 2
    partials = pl.pallas_call(
        _loss_body,
        grid=(2, g2),
        in_specs=[
            pl.BlockSpec((_BH, d), lambda i, j: (2 * (i * g2 + j), 0)),
            pl.BlockSpec((_BH, d), lambda i, j: (2 * (i * g2 + j) + 1, 0)),
            pl.BlockSpec((_BH, m), lambda i, j: (2 * (i * g2 + j), 0)),
            pl.BlockSpec((_BH, m), lambda i, j: (2 * (i * g2 + j) + 1, 0)),
            pl.BlockSpec((m, d), lambda i, j: (0, 0)),
        ],
        out_specs=pl.BlockSpec((1, 1, m), lambda i, j: (i * g2 + j, 0, 0)),
        out_shape=jax.ShapeDtypeStruct((g, 1, m), jnp.float32),
        compiler_params=pltpu.CompilerParams(
            dimension_semantics=("parallel", "arbitrary"),
        ),
    )(X, X, r, r, mus)
    return jnp.sum(partials) / n
